# baseline (speedup 1.0000x reference)
import jax
import jax.numpy as jnp
from jax import lax
from jax.experimental import pallas as pl
from jax.experimental.pallas import tpu as pltpu
from jax.experimental.pallas import tpu_sc as plsc

N_PIX = 262144
K = 8192
NSUB = 16
L = 16
EPT = N_PIX // NSUB
TRK = K // NSUB
NVREG = EPT // L
OUT_PAD = K + N_PIX
R, C = 64, 128


def _sc_body(err_hbm, idx_hbm, oerr_hbm, oidx_hbm,
             upd_hbm, serr_hbm, spos_hbm, sidx_hbm,
             s_sh, stage_sh, gmax_sh, hstage_sh, cnt_sh,
             idx_v, err_v, buf_v, dest_v, h2d_v, h1d_v, gh_v,
             smax_v, out_v, oi_v, oslot_v, oerr_v, cnt_v,
             cnt2d_v, sem):
    c = lax.axis_index("c")
    s = lax.axis_index("s")
    lane = lax.iota(jnp.int32, L)
    shard = s * EPT

    pltpu.sync_copy(idx_hbm.at[pl.ds(shard, EPT)], idx_v)
    pltpu.sync_copy(err_hbm.at[pl.ds(shard, EPT)], err_v)

    @pl.when(c == 0)
    def _join():
        def fneg(i, _):
            oslot_v[pl.ds(i * L, L)] = jnp.full((L,), -1, jnp.int32)
            return 0
        lax.fori_loop(0, TRK // L, fneg, 0)

        def fcopy(h, _):
            pltpu.sync_copy(oslot_v, s_sh.at[pl.ds(s * EPT + h * TRK, TRK)])
            return 0
        lax.fori_loop(0, EPT // TRK, fcopy, 0)
        plsc.subcore_barrier()

        pltpu.sync_copy(oidx_hbm.at[pl.ds(s * TRK, TRK)], oi_v)

        def fslot(i, _):
            oslot_v[pl.ds(i * L, L)] = s * TRK + i * L + lane
            return 0
        lax.fori_loop(0, TRK // L, fslot, 0)
        pltpu.async_copy(oslot_v, s_sh.at[oi_v], sem).wait()
        plsc.subcore_barrier()

        pltpu.async_copy(s_sh.at[idx_v], buf_v, sem).wait()

        def fzero(i, _):
            smax_v[pl.ds(i * L, L)] = jnp.zeros((L,), jnp.float32)
            return 0
        lax.fori_loop(0, K // L, fzero, 0)

        def accum(i, _):
            sl = buf_v[pl.ds(i * L, L)]
            ev = err_v[pl.ds(i * L, L)]
            hit = sl >= 0
            ks = jnp.where(hit, sl, 0)

            def cond(carry):
                return jnp.any(carry[0])

            def body(carry):
                rem, it = carry
                cur = plsc.load_gather(smax_v, [ks], mask=rem)
                newv = jnp.maximum(cur, ev)
                plsc.store_scatter(smax_v, [ks], newv, mask=rem)
                back = plsc.load_gather(smax_v, [ks], mask=rem)
                return rem & (back < newv), it + jnp.int32(1)

            lax.while_loop(cond, body, (hit, jnp.int32(0)))
            return 0
        lax.fori_loop(0, NVREG, accum, 0)

        plsc.subcore_barrier()
        pltpu.sync_copy(smax_v, stage_sh.at[pl.ds(s * K, K)])
        plsc.subcore_barrier()
        pltpu.sync_copy(stage_sh.at[pl.ds(s * TRK, TRK)], out_v)

        def rmerge(r, _):
            pltpu.sync_copy(stage_sh.at[pl.ds(r * K + s * TRK, TRK)], oerr_v)

            def red(i, _):
                out_v[pl.ds(i * L, L)] = jnp.maximum(
                    out_v[pl.ds(i * L, L)], oerr_v[pl.ds(i * L, L)])
                return 0
            lax.fori_loop(0, TRK // L, red, 0)
            return 0
        lax.fori_loop(1, NSUB, rmerge, 0)
        pltpu.sync_copy(out_v, gmax_sh.at[pl.ds(s * TRK, TRK)])
        plsc.subcore_barrier()

        pltpu.sync_copy(gmax_sh, smax_v)
        pltpu.sync_copy(oerr_hbm.at[pl.ds(s * TRK, TRK)], oerr_v)
        pltpu.async_copy(s_sh.at[oi_v], oslot_v, sem).wait()

        def answer(i, _):
            sl = oslot_v[pl.ds(i * L, L)]
            g = plsc.load_gather(smax_v, [sl])
            out_v[pl.ds(i * L, L)] = jnp.maximum(oerr_v[pl.ds(i * L, L)], g)
            return 0
        lax.fori_loop(0, TRK // L, answer, 0)
        pltpu.sync_copy(out_v, upd_hbm.at[pl.ds(s * TRK, TRK)])

    @pl.when(c == 1)
    def _select():
        def one_pass(p, carry):
            prefix, k_rem = carry
            shift = 20 - 10 * p

            def hz(i, _):
                for r in range(NSUB):
                    h2d_v[pl.ds(r * 1024 + i * L, L)] = jnp.zeros((L,), jnp.int32)
                return 0
            lax.fori_loop(0, 1024 // L, hz, 0)

            def scan(i, _):
                b = plsc.bitcast(err_v[pl.ds(i * L, L)], jnp.int32)
                shv = jnp.full((L,), shift, jnp.int32)
                d = jnp.bitwise_and(lax.shift_right_logical(b, shv),
                                    jnp.full((L,), 1023, jnp.int32))
                hi = lax.shift_right_logical(b, shv + 10)
                m = hi == jnp.full((L,), lax.shift_right_logical(
                    prefix, shift + 10), jnp.int32)
                plsc.addupdate_scatter(h2d_v, [lane * 1024 + d],
                                       jnp.full((L,), 1, jnp.int32), mask=m)
                return 0
            lax.fori_loop(0, NVREG, scan, 0)

            def lm(i, _):
                acc = h2d_v[pl.ds(i * L, L)]
                for r in range(1, NSUB):
                    acc = acc + h2d_v[pl.ds(r * 1024 + i * L, L)]
                h1d_v[pl.ds(i * L, L)] = acc
                return 0
            lax.fori_loop(0, 1024 // L, lm, 0)

            pltpu.sync_copy(h1d_v, hstage_sh.at[pl.ds(s * 1024, 1024)])
            plsc.subcore_barrier()
            pltpu.sync_copy(hstage_sh, h2d_v)

            def tm(i, _):
                acc = h2d_v[pl.ds(i * L, L)]
                for r in range(1, NSUB):
                    acc = acc + h2d_v[pl.ds(r * 1024 + i * L, L)]
                gh_v[pl.ds(i * L, L)] = acc
                return 0
            lax.fori_loop(0, 1024 // L, tm, 0)
            plsc.subcore_barrier()

            def sscan(i, carry2):
                sfx_c, d_acc, nk_acc = carry2
                v = 63 - i
                h = gh_v[pl.ds(v * L, L)]
                sfx_incl = lax.rev(plsc.cumsum(lax.rev(h, (0,))), (0,))
                sfx = sfx_incl - h + sfx_c
                cond = (sfx < k_rem) & (sfx + h >= k_rem)
                tg = v * L + lane
                d_acc = d_acc + jnp.sum(jnp.where(cond, tg, 0))
                nk_acc = nk_acc + jnp.sum(jnp.where(cond, k_rem - sfx, 0))
                return sfx_c + jnp.sum(h), d_acc, nk_acc
            _, dig, newk = lax.fori_loop(
                0, 1024 // L, sscan,
                (jnp.int32(0), jnp.int32(0), jnp.int32(0)))
            return prefix | lax.shift_left(dig, shift), newk

        t_bits, need = lax.fori_loop(0, 3, one_pass,
                                     (jnp.int32(0), jnp.int32(K)))
        cnt_gt_total = K - need

        def csweep(i, carry2):
            cg, ce = carry2
            b = plsc.bitcast(err_v[pl.ds(i * L, L)], jnp.int32)
            tb = jnp.full((L,), t_bits, jnp.int32)
            cg = cg + jnp.sum((b > tb).astype(jnp.int32))
            ce = ce + jnp.sum((b == tb).astype(jnp.int32))
            return cg, ce
        cgt, ceq = lax.fori_loop(0, NVREG, csweep,
                                 (jnp.int32(0), jnp.int32(0)))
        cnt_v[...] = jnp.where(lane == 0, cgt,
                               jnp.where(lane == 1, ceq, 0))
        pltpu.sync_copy(cnt_v, cnt_sh.at[pl.ds(s * L, L)])
        plsc.subcore_barrier()
        pltpu.sync_copy(cnt_sh, cnt2d_v)

        def bases(t, carry2):
            gb, eb = carry2
            row = cnt2d_v[pl.ds(t * L, L)]
            take = (t < s).astype(jnp.int32)
            gb = gb + take * jnp.sum(jnp.where(lane == 0, row, 0))
            eb = eb + take * jnp.sum(jnp.where(lane == 1, row, 0))
            return gb, eb
        gt_base, eq_base = lax.fori_loop(0, NSUB, bases,
                                         (jnp.int32(0), jnp.int32(0)))

        def dsweep(i, carry2):
            rg, re = carry2
            b = plsc.bitcast(err_v[pl.ds(i * L, L)], jnp.int32)
            tb = jnp.full((L,), t_bits, jnp.int32)
            m_gt = b > tb
            m_eq = b == tb
            r_gt = plsc.cumsum(m_gt.astype(jnp.int32))
            r_eq = plsc.cumsum(m_eq.astype(jnp.int32))
            pos = shard + i * L + lane
            dgt = gt_base + rg + r_gt - 1
            der = eq_base + re + r_eq - 1
            kept = m_eq & (der < need)
            dest = jnp.where(m_gt, dgt,
                             jnp.where(kept, cnt_gt_total + der, K + pos))
            dest_v[pl.ds(i * L, L)] = dest
            buf_v[pl.ds(i * L, L)] = pos
            return (rg + jnp.sum(m_gt.astype(jnp.int32)),
                    re + jnp.sum(m_eq.astype(jnp.int32)))
        lax.fori_loop(0, NVREG, dsweep, (jnp.int32(0), jnp.int32(0)))

        pltpu.async_copy(err_v, serr_hbm.at[dest_v], sem).wait()
        pltpu.async_copy(buf_v, spos_hbm.at[dest_v], sem).wait()
        pltpu.async_copy(idx_v, sidx_hbm.at[dest_v], sem).wait()


def _sc_call(errors, indices, old_errors, old_indices):
    mesh = plsc.VectorSubcoreMesh(core_axis_name="c", subcore_axis_name="s",
                                  num_cores=2, num_subcores=NSUB)
    f = pl.kernel(
        _sc_body,
        mesh=mesh,
        out_type=(
            jax.ShapeDtypeStruct((K,), jnp.float32),
            jax.ShapeDtypeStruct((OUT_PAD,), jnp.float32),
            jax.ShapeDtypeStruct((OUT_PAD,), jnp.int32),
            jax.ShapeDtypeStruct((OUT_PAD,), jnp.int32),
        ),
        compiler_params=pltpu.CompilerParams(needs_layout_passes=False),
        scratch_types=[
            pltpu.VMEM_SHARED((N_PIX,), jnp.int32),
            pltpu.VMEM_SHARED((NSUB * K,), jnp.float32),
            pltpu.VMEM_SHARED((K,), jnp.float32),
            pltpu.VMEM_SHARED((NSUB * 1024,), jnp.int32),
            pltpu.VMEM_SHARED((NSUB * L,), jnp.int32),
            pltpu.VMEM((EPT,), jnp.int32),
            pltpu.VMEM((EPT,), jnp.float32),
            pltpu.VMEM((EPT,), jnp.int32),
            pltpu.VMEM((EPT,), jnp.int32),
            pltpu.VMEM((NSUB * 1024,), jnp.int32),
            pltpu.VMEM((1024,), jnp.int32),
            pltpu.VMEM((1024,), jnp.int32),
            pltpu.VMEM((K,), jnp.float32),
            pltpu.VMEM((TRK,), jnp.float32),
            pltpu.VMEM((TRK,), jnp.int32),
            pltpu.VMEM((TRK,), jnp.int32),
            pltpu.VMEM((TRK,), jnp.float32),
            pltpu.VMEM((L,), jnp.int32),
            pltpu.VMEM((NSUB * L,), jnp.int32),
            pltpu.SemaphoreType.DMA,
        ],
    )
    return f(errors, indices, old_errors, old_indices)


def _lin_iota():
    r = lax.broadcasted_iota(jnp.int32, (R, C), 0)
    c = lax.broadcasted_iota(jnp.int32, (R, C), 1)
    return r * C + c


def _xor_shuffle(x, d):
    if d < C:
        fwd = pltpu.roll(x, C - d, 1)
        bwd = pltpu.roll(x, d, 1)
        sel = (lax.broadcasted_iota(jnp.int32, (R, C), 1) & d) == 0
    else:
        dr = d // C
        fwd = pltpu.roll(x, R - dr, 0)
        bwd = pltpu.roll(x, dr, 0)
        sel = (lax.broadcasted_iota(jnp.int32, (R, C), 0) & dr) == 0
    return jnp.where(sel, fwd, bwd)


def _bitonic3(key, pos, idx):
    e = _lin_iota()
    for k in range(1, 14):
        for j in range(k - 1, -1, -1):
            d = 1 << j
            kk = _xor_shuffle(key, d)
            pp = _xor_shuffle(pos, d)
            ii = _xor_shuffle(idx, d)
            upper = (e & d) != 0
            if k < 13:
                asc = (e & (1 << k)) != 0
            else:
                asc = jnp.zeros_like(upper)
            keep_larger = ~(upper ^ asc)
            mine_gt = (key > kk) | ((key == kk) & (pos < pp))
            take_mine = ~(keep_larger ^ mine_gt)
            key = jnp.where(take_mine, key, kk)
            pos = jnp.where(take_mine, pos, pp)
            idx = jnp.where(take_mine, idx, ii)
    return key, pos, idx


def _bitonic1(v):
    e = _lin_iota()
    for k in range(1, 14):
        for j in range(k - 1, -1, -1):
            d = 1 << j
            vv = _xor_shuffle(v, d)
            upper = (e & d) != 0
            if k < 13:
                asc = (e & (1 << k)) != 0
            else:
                asc = jnp.zeros_like(upper)
            keep_larger = ~(upper ^ asc)
            v = jnp.where(keep_larger, jnp.maximum(v, vv), jnp.minimum(v, vv))
    return v


def _sort_merge_body(se_ref, sp_ref, si_ref, upd_ref, oi_ref, oe_ref, ooi_ref):
    key, pos, idx = _bitonic3(se_ref[...], sp_ref[...], si_ref[...])
    me = _bitonic1(upd_ref[...])
    surpassed = key > me
    oe_ref[...] = jnp.where(surpassed, key, me)
    ooi_ref[...] = jnp.where(surpassed, idx, oi_ref[...])


def _tc_call(sel_err, sel_pos, sel_idx, updated, old_idx):
    return pl.pallas_call(
        _sort_merge_body,
        out_shape=(
            jax.ShapeDtypeStruct((R, C), jnp.float32),
            jax.ShapeDtypeStruct((R, C), jnp.int32),
        ),
    )(sel_err, sel_pos, sel_idx, updated, old_idx)


def kernel(errors, indices, old_errors, old_indices):
    errors_flat = errors.reshape(-1)
    indices_flat = indices.reshape(-1)
    upd, sel_err, sel_pos, sel_idx = _sc_call(
        errors_flat, indices_flat, old_errors, old_indices)
    oe, oi = _tc_call(
        sel_err[:K].reshape(R, C), sel_pos[:K].reshape(R, C),
        sel_idx[:K].reshape(R, C), upd.reshape(R, C),
        old_indices.reshape(R, C))
    return oe.reshape(K), oi.reshape(K)


# speedup vs baseline: 1.0546x; 1.0546x over previous
import jax
import jax.numpy as jnp
from jax import lax
from jax.experimental import pallas as pl
from jax.experimental.pallas import tpu as pltpu
from jax.experimental.pallas import tpu_sc as plsc

N_PIX = 262144
K = 8192
NSUB = 16
L = 16
EPT = N_PIX // NSUB
TRK = K // NSUB
NVREG = EPT // L
OUT_PAD = K + N_PIX
R, C = 64, 128


def _sc_body(err_hbm, idx_hbm, oerr_hbm, oidx_hbm,
             upd_hbm, serr_hbm, spos_hbm, sidx_hbm,
             s_sh, stage_sh, gmax_sh, hstage_sh, cnt_sh,
             idx_v, err_v, buf_v, dest_v, h2d_v, h1d_v, gh_v,
             smax_v, out_v, oi_v, oslot_v, oerr_v, cnt_v,
             cnt2d_v, sem):
    c = lax.axis_index("c")
    s = lax.axis_index("s")
    lane = lax.iota(jnp.int32, L)
    shard = s * EPT

    pltpu.sync_copy(idx_hbm.at[pl.ds(shard, EPT)], idx_v)
    pltpu.sync_copy(err_hbm.at[pl.ds(shard, EPT)], err_v)

    @pl.when(c == 0)
    def _join():
      with jax.named_scope("join_setup"):
        def fneg(i, _):
            oslot_v[pl.ds(i * L, L)] = jnp.full((L,), -1, jnp.int32)
            return 0
        lax.fori_loop(0, TRK // L, fneg, 0)

        def fcopy(h, _):
            pltpu.sync_copy(oslot_v, s_sh.at[pl.ds(s * EPT + h * TRK, TRK)])
            return 0
        lax.fori_loop(0, EPT // TRK, fcopy, 0)
        plsc.subcore_barrier()

        pltpu.sync_copy(oidx_hbm.at[pl.ds(s * TRK, TRK)], oi_v)

        def fslot(i, _):
            oslot_v[pl.ds(i * L, L)] = s * TRK + i * L + lane
            return 0
        lax.fori_loop(0, TRK // L, fslot, 0)
        pltpu.async_copy(oslot_v, s_sh.at[oi_v], sem).wait()
        plsc.subcore_barrier()

        pltpu.async_copy(s_sh.at[idx_v], buf_v, sem).wait()

        def fzero(i, _):
            smax_v[pl.ds(i * L, L)] = jnp.zeros((L,), jnp.float32)
            return 0
        lax.fori_loop(0, K // L, fzero, 0)

      with jax.named_scope("join_accum"):
        def accum(i, _):
            sl = buf_v[pl.ds(i * L, L)]
            ev = err_v[pl.ds(i * L, L)]
            hit = sl >= 0
            ks = jnp.where(hit, sl, 0)

            def cond(carry):
                return jnp.any(carry[0])

            def body(carry):
                rem, it = carry
                cur = plsc.load_gather(smax_v, [ks], mask=rem)
                newv = jnp.maximum(cur, ev)
                plsc.store_scatter(smax_v, [ks], newv, mask=rem)
                back = plsc.load_gather(smax_v, [ks], mask=rem)
                return rem & (back < newv), it + jnp.int32(1)

            lax.while_loop(cond, body, (hit, jnp.int32(0)))
            return 0
        lax.fori_loop(0, NVREG, accum, 0)

      with jax.named_scope("join_merge"):
        plsc.subcore_barrier()
        pltpu.sync_copy(smax_v, stage_sh.at[pl.ds(s * K, K)])
        plsc.subcore_barrier()
        pltpu.sync_copy(stage_sh.at[pl.ds(s * TRK, TRK)], out_v)

        def rmerge(r, _):
            pltpu.sync_copy(stage_sh.at[pl.ds(r * K + s * TRK, TRK)], oerr_v)

            def red(i, _):
                out_v[pl.ds(i * L, L)] = jnp.maximum(
                    out_v[pl.ds(i * L, L)], oerr_v[pl.ds(i * L, L)])
                return 0
            lax.fori_loop(0, TRK // L, red, 0)
            return 0
        lax.fori_loop(1, NSUB, rmerge, 0)
        pltpu.sync_copy(out_v, gmax_sh.at[pl.ds(s * TRK, TRK)])
        plsc.subcore_barrier()

      with jax.named_scope("join_answer"):
        pltpu.sync_copy(gmax_sh, smax_v)
        pltpu.sync_copy(oerr_hbm.at[pl.ds(s * TRK, TRK)], oerr_v)
        pltpu.async_copy(s_sh.at[oi_v], oslot_v, sem).wait()

        def answer(i, _):
            sl = oslot_v[pl.ds(i * L, L)]
            g = plsc.load_gather(smax_v, [sl])
            out_v[pl.ds(i * L, L)] = jnp.maximum(oerr_v[pl.ds(i * L, L)], g)
            return 0
        lax.fori_loop(0, TRK // L, answer, 0)
        pltpu.sync_copy(out_v, upd_hbm.at[pl.ds(s * TRK, TRK)])

    @pl.when(c == 1)
    def _select():
      with jax.named_scope("sel_passes"):
        def one_pass(p, carry):
            prefix, k_rem = carry
            shift = 20 - 10 * p

            def hz(i, _):
                for r in range(NSUB):
                    h2d_v[pl.ds(r * 1024 + i * L, L)] = jnp.zeros((L,), jnp.int32)
                return 0
            lax.fori_loop(0, 1024 // L, hz, 0)

            def scan(i, _):
                b = plsc.bitcast(err_v[pl.ds(i * L, L)], jnp.int32)
                shv = jnp.full((L,), shift, jnp.int32)
                d = jnp.bitwise_and(lax.shift_right_logical(b, shv),
                                    jnp.full((L,), 1023, jnp.int32))
                hi = lax.shift_right_logical(b, shv + 10)
                m = hi == jnp.full((L,), lax.shift_right_logical(
                    prefix, shift + 10), jnp.int32)
                plsc.addupdate_scatter(h2d_v, [lane * 1024 + d],
                                       jnp.full((L,), 1, jnp.int32), mask=m)
                return 0
            lax.fori_loop(0, NVREG, scan, 0)

            def lm(i, _):
                acc = h2d_v[pl.ds(i * L, L)]
                for r in range(1, NSUB):
                    acc = acc + h2d_v[pl.ds(r * 1024 + i * L, L)]
                h1d_v[pl.ds(i * L, L)] = acc
                return 0
            lax.fori_loop(0, 1024 // L, lm, 0)

            pltpu.sync_copy(h1d_v, hstage_sh.at[pl.ds(s * 1024, 1024)])
            plsc.subcore_barrier()
            pltpu.sync_copy(hstage_sh, h2d_v)

            def tm(i, _):
                acc = h2d_v[pl.ds(i * L, L)]
                for r in range(1, NSUB):
                    acc = acc + h2d_v[pl.ds(r * 1024 + i * L, L)]
                gh_v[pl.ds(i * L, L)] = acc
                return 0
            lax.fori_loop(0, 1024 // L, tm, 0)
            plsc.subcore_barrier()

            def sscan(i, carry2):
                sfx_c, d_acc, nk_acc = carry2
                v = 63 - i
                h = gh_v[pl.ds(v * L, L)]
                sfx_incl = lax.rev(plsc.cumsum(lax.rev(h, (0,))), (0,))
                sfx = sfx_incl - h + sfx_c
                cond = (sfx < k_rem) & (sfx + h >= k_rem)
                tg = v * L + lane
                d_acc = d_acc + jnp.sum(jnp.where(cond, tg, 0))
                nk_acc = nk_acc + jnp.sum(jnp.where(cond, k_rem - sfx, 0))
                return sfx_c + jnp.sum(h), d_acc, nk_acc
            _, dig, newk = lax.fori_loop(
                0, 1024 // L, sscan,
                (jnp.int32(0), jnp.int32(0), jnp.int32(0)))
            return prefix | lax.shift_left(dig, shift), newk

        t_bits, need = lax.fori_loop(0, 3, one_pass,
                                     (jnp.int32(0), jnp.int32(K)))
        cnt_gt_total = K - need

      with jax.named_scope("sel_counts"):
        def csweep(i, carry2):
            cg, ce = carry2
            b = plsc.bitcast(err_v[pl.ds(i * L, L)], jnp.int32)
            tb = jnp.full((L,), t_bits, jnp.int32)
            cg = cg + jnp.sum((b > tb).astype(jnp.int32))
            ce = ce + jnp.sum((b == tb).astype(jnp.int32))
            return cg, ce
        cgt, ceq = lax.fori_loop(0, NVREG, csweep,
                                 (jnp.int32(0), jnp.int32(0)))
        cnt_v[...] = jnp.where(lane == 0, cgt,
                               jnp.where(lane == 1, ceq, 0))
        pltpu.sync_copy(cnt_v, cnt_sh.at[pl.ds(s * L, L)])
        plsc.subcore_barrier()
        pltpu.sync_copy(cnt_sh, cnt2d_v)

        def bases(t, carry2):
            gb, eb = carry2
            row = cnt2d_v[pl.ds(t * L, L)]
            take = (t < s).astype(jnp.int32)
            gb = gb + take * jnp.sum(jnp.where(lane == 0, row, 0))
            eb = eb + take * jnp.sum(jnp.where(lane == 1, row, 0))
            return gb, eb
        gt_base, eq_base = lax.fori_loop(0, NSUB, bases,
                                         (jnp.int32(0), jnp.int32(0)))

      with jax.named_scope("sel_dsweep"):
        def dsweep(i, carry2):
            rg, re = carry2
            b = plsc.bitcast(err_v[pl.ds(i * L, L)], jnp.int32)
            tb = jnp.full((L,), t_bits, jnp.int32)
            m_gt = b > tb
            m_eq = b == tb
            r_gt = plsc.cumsum(m_gt.astype(jnp.int32))
            r_eq = plsc.cumsum(m_eq.astype(jnp.int32))
            pos = shard + i * L + lane
            dgt = gt_base + rg + r_gt - 1
            der = eq_base + re + r_eq - 1
            kept = m_eq & (der < need)
            dest = jnp.where(m_gt, dgt,
                             jnp.where(kept, cnt_gt_total + der, K + pos))
            dest_v[pl.ds(i * L, L)] = dest
            buf_v[pl.ds(i * L, L)] = pos
            return (rg + jnp.sum(m_gt.astype(jnp.int32)),
                    re + jnp.sum(m_eq.astype(jnp.int32)))
        lax.fori_loop(0, NVREG, dsweep, (jnp.int32(0), jnp.int32(0)))

      with jax.named_scope("sel_scatter"):
        pltpu.async_copy(err_v, serr_hbm.at[dest_v], sem).wait()
        pltpu.async_copy(buf_v, spos_hbm.at[dest_v], sem).wait()
        pltpu.async_copy(idx_v, sidx_hbm.at[dest_v], sem).wait()


def _sc_call(errors, indices, old_errors, old_indices):
    mesh = plsc.VectorSubcoreMesh(core_axis_name="c", subcore_axis_name="s",
                                  num_cores=2, num_subcores=NSUB)
    f = pl.kernel(
        _sc_body,
        mesh=mesh,
        out_type=(
            jax.ShapeDtypeStruct((K,), jnp.float32),
            jax.ShapeDtypeStruct((OUT_PAD,), jnp.float32),
            jax.ShapeDtypeStruct((OUT_PAD,), jnp.int32),
            jax.ShapeDtypeStruct((OUT_PAD,), jnp.int32),
        ),
        compiler_params=pltpu.CompilerParams(needs_layout_passes=False),
        scratch_types=[
            pltpu.VMEM_SHARED((N_PIX,), jnp.int32),
            pltpu.VMEM_SHARED((NSUB * K,), jnp.float32),
            pltpu.VMEM_SHARED((K,), jnp.float32),
            pltpu.VMEM_SHARED((NSUB * 1024,), jnp.int32),
            pltpu.VMEM_SHARED((NSUB * L,), jnp.int32),
            pltpu.VMEM((EPT,), jnp.int32),
            pltpu.VMEM((EPT,), jnp.float32),
            pltpu.VMEM((EPT,), jnp.int32),
            pltpu.VMEM((EPT,), jnp.int32),
            pltpu.VMEM((NSUB * 1024,), jnp.int32),
            pltpu.VMEM((1024,), jnp.int32),
            pltpu.VMEM((1024,), jnp.int32),
            pltpu.VMEM((K,), jnp.float32),
            pltpu.VMEM((TRK,), jnp.float32),
            pltpu.VMEM((TRK,), jnp.int32),
            pltpu.VMEM((TRK,), jnp.int32),
            pltpu.VMEM((TRK,), jnp.float32),
            pltpu.VMEM((L,), jnp.int32),
            pltpu.VMEM((NSUB * L,), jnp.int32),
            pltpu.SemaphoreType.DMA,
        ],
    )
    return f(errors, indices, old_errors, old_indices)


def _lin_iota():
    r = lax.broadcasted_iota(jnp.int32, (R, C), 0)
    c = lax.broadcasted_iota(jnp.int32, (R, C), 1)
    return r * C + c


def _xor_shuffle(x, d):
    if d < C:
        fwd = pltpu.roll(x, C - d, 1)
        bwd = pltpu.roll(x, d, 1)
        sel = (lax.broadcasted_iota(jnp.int32, (R, C), 1) & d) == 0
    else:
        dr = d // C
        fwd = pltpu.roll(x, R - dr, 0)
        bwd = pltpu.roll(x, dr, 0)
        sel = (lax.broadcasted_iota(jnp.int32, (R, C), 0) & dr) == 0
    return jnp.where(sel, fwd, bwd)


def _bitonic3(key, pos, idx):
    e = _lin_iota()
    for k in range(1, 14):
        for j in range(k - 1, -1, -1):
            d = 1 << j
            kk = _xor_shuffle(key, d)
            pp = _xor_shuffle(pos, d)
            ii = _xor_shuffle(idx, d)
            upper = (e & d) != 0
            if k < 13:
                asc = (e & (1 << k)) != 0
            else:
                asc = jnp.zeros_like(upper)
            keep_larger = ~(upper ^ asc)
            mine_gt = (key > kk) | ((key == kk) & (pos < pp))
            take_mine = ~(keep_larger ^ mine_gt)
            key = jnp.where(take_mine, key, kk)
            pos = jnp.where(take_mine, pos, pp)
            idx = jnp.where(take_mine, idx, ii)
    return key, pos, idx


def _bitonic1(v):
    e = _lin_iota()
    for k in range(1, 14):
        for j in range(k - 1, -1, -1):
            d = 1 << j
            vv = _xor_shuffle(v, d)
            upper = (e & d) != 0
            if k < 13:
                asc = (e & (1 << k)) != 0
            else:
                asc = jnp.zeros_like(upper)
            keep_larger = ~(upper ^ asc)
            v = jnp.where(keep_larger, jnp.maximum(v, vv), jnp.minimum(v, vv))
    return v


def _sort_merge_body(se_ref, sp_ref, si_ref, upd_ref, oi_ref, oe_ref, ooi_ref):
    key, pos, idx = _bitonic3(se_ref[...], sp_ref[...], si_ref[...])
    me = _bitonic1(upd_ref[...])
    surpassed = key > me
    oe_ref[...] = jnp.where(surpassed, key, me)
    ooi_ref[...] = jnp.where(surpassed, idx, oi_ref[...])


def _tc_call(sel_err, sel_pos, sel_idx, updated, old_idx):
    return pl.pallas_call(
        _sort_merge_body,
        out_shape=(
            jax.ShapeDtypeStruct((R, C), jnp.float32),
            jax.ShapeDtypeStruct((R, C), jnp.int32),
        ),
    )(sel_err, sel_pos, sel_idx, updated, old_idx)


def kernel(errors, indices, old_errors, old_indices):
    errors_flat = errors.reshape(-1)
    indices_flat = indices.reshape(-1)
    upd, sel_err, sel_pos, sel_idx = _sc_call(
        errors_flat, indices_flat, old_errors, old_indices)
    oe, oi = _tc_call(
        sel_err[:K].reshape(R, C), sel_pos[:K].reshape(R, C),
        sel_idx[:K].reshape(R, C), upd.reshape(R, C),
        old_indices.reshape(R, C))
    return oe.reshape(K), oi.reshape(K)


# speedup vs baseline: 22.1024x; 20.9572x over previous
import jax
import jax.numpy as jnp
from jax import lax
from jax.experimental import pallas as pl
from jax.experimental.pallas import tpu as pltpu
from jax.experimental.pallas import tpu_sc as plsc

N_PIX = 262144
K = 8192
NSUB = 16
L = 16
EPT = N_PIX // NSUB
TRK = K // NSUB
NVREG = EPT // L
CAP = 1024
W = NSUB * CAP
OUT_PAD = W + N_PIX
R, C = 64, 128
R2 = 128


def _sc_body(err_hbm, idx_hbm, oerr_hbm, oidx_hbm,
             upd_hbm, serr_hbm, spos_hbm, sidx_hbm,
             s_sh, stage_sh, gmax_sh, hstage_sh, cnt_sh,
             idx_v, err_v, buf_v, dest_v, h2d_v, h1d_v, gh_v,
             smax_v, out_v, oi_v, oslot_v, oerr_v, cnt_v,
             cnt2d_v, werr_v, wpos_v, widx_v, sem):
    c = lax.axis_index("c")
    s = lax.axis_index("s")
    lane = lax.iota(jnp.int32, L)
    shard = s * EPT

    pltpu.sync_copy(idx_hbm.at[pl.ds(shard, EPT)], idx_v)
    pltpu.sync_copy(err_hbm.at[pl.ds(shard, EPT)], err_v)

    @pl.when(c == 0)
    def _join():
      with jax.named_scope("join_setup"):
        def fneg(i, _):
            oslot_v[pl.ds(i * L, L)] = jnp.full((L,), -1, jnp.int32)
            return 0
        lax.fori_loop(0, TRK // L, fneg, 0)

        def fcopy(h, _):
            pltpu.sync_copy(oslot_v, s_sh.at[pl.ds(s * EPT + h * TRK, TRK)])
            return 0
        lax.fori_loop(0, EPT // TRK, fcopy, 0)
        plsc.subcore_barrier()

        pltpu.sync_copy(oidx_hbm.at[pl.ds(s * TRK, TRK)], oi_v)

        def fslot(i, _):
            oslot_v[pl.ds(i * L, L)] = s * TRK + i * L + lane
            return 0
        lax.fori_loop(0, TRK // L, fslot, 0)
        pltpu.async_copy(oslot_v, s_sh.at[oi_v], sem).wait()
        plsc.subcore_barrier()

        pltpu.async_copy(s_sh.at[idx_v], buf_v, sem).wait()

        def fzero(i, _):
            smax_v[pl.ds(i * L, L)] = jnp.zeros((L,), jnp.float32)
            return 0
        lax.fori_loop(0, K // L, fzero, 0)

      with jax.named_scope("join_accum"):
        def accum(i, _):
            sl = buf_v[pl.ds(i * L, L)]
            ev = err_v[pl.ds(i * L, L)]
            hit = sl >= 0
            ks = jnp.where(hit, sl, 0)

            def cond(carry):
                return jnp.any(carry[0])

            def body(carry):
                rem, it = carry
                cur = plsc.load_gather(smax_v, [ks], mask=rem)
                newv = jnp.maximum(cur, ev)
                plsc.store_scatter(smax_v, [ks], newv, mask=rem)
                back = plsc.load_gather(smax_v, [ks], mask=rem)
                return rem & (back < newv), it + jnp.int32(1)

            lax.while_loop(cond, body, (hit, jnp.int32(0)))
            return 0
        lax.fori_loop(0, NVREG, accum, 0)

      with jax.named_scope("join_merge"):
        plsc.subcore_barrier()
        pltpu.sync_copy(smax_v, stage_sh.at[pl.ds(s * K, K)])
        plsc.subcore_barrier()
        pltpu.sync_copy(stage_sh.at[pl.ds(s * TRK, TRK)], out_v)

        def rmerge(r, _):
            pltpu.sync_copy(stage_sh.at[pl.ds(r * K + s * TRK, TRK)], oerr_v)

            def red(i, _):
                out_v[pl.ds(i * L, L)] = jnp.maximum(
                    out_v[pl.ds(i * L, L)], oerr_v[pl.ds(i * L, L)])
                return 0
            lax.fori_loop(0, TRK // L, red, 0)
            return 0
        lax.fori_loop(1, NSUB, rmerge, 0)
        pltpu.sync_copy(out_v, gmax_sh.at[pl.ds(s * TRK, TRK)])
        plsc.subcore_barrier()

      with jax.named_scope("join_answer"):
        pltpu.sync_copy(gmax_sh, smax_v)
        pltpu.sync_copy(oerr_hbm.at[pl.ds(s * TRK, TRK)], oerr_v)
        pltpu.async_copy(s_sh.at[oi_v], oslot_v, sem).wait()

        def answer(i, _):
            sl = oslot_v[pl.ds(i * L, L)]
            g = plsc.load_gather(smax_v, [sl])
            out_v[pl.ds(i * L, L)] = jnp.maximum(oerr_v[pl.ds(i * L, L)], g)
            return 0
        lax.fori_loop(0, TRK // L, answer, 0)
        pltpu.sync_copy(out_v, upd_hbm.at[pl.ds(s * TRK, TRK)])

    @pl.when(c == 1)
    def _select():
      with jax.named_scope("sel_passes"):
        def one_pass(p, carry):
            prefix, k_rem = carry
            shift = 20 - 10 * p

            def hz(i, _):
                for r in range(NSUB):
                    h2d_v[pl.ds(r * 1024 + i * L, L)] = jnp.zeros((L,), jnp.int32)
                return 0
            lax.fori_loop(0, 1024 // L, hz, 0)

            def scan(i, _):
                b = plsc.bitcast(err_v[pl.ds(i * L, L)], jnp.int32)
                shv = jnp.full((L,), shift, jnp.int32)
                d = jnp.bitwise_and(lax.shift_right_logical(b, shv),
                                    jnp.full((L,), 1023, jnp.int32))
                hi = lax.shift_right_logical(b, shv + 10)
                m = hi == jnp.full((L,), lax.shift_right_logical(
                    prefix, shift + 10), jnp.int32)
                plsc.addupdate_scatter(h2d_v, [lane * 1024 + d],
                                       jnp.full((L,), 1, jnp.int32), mask=m)
                return 0
            lax.fori_loop(0, NVREG, scan, 0)

            def lm(i, _):
                acc = h2d_v[pl.ds(i * L, L)]
                for r in range(1, NSUB):
                    acc = acc + h2d_v[pl.ds(r * 1024 + i * L, L)]
                h1d_v[pl.ds(i * L, L)] = acc
                return 0
            lax.fori_loop(0, 1024 // L, lm, 0)

            pltpu.sync_copy(h1d_v, hstage_sh.at[pl.ds(s * 1024, 1024)])
            plsc.subcore_barrier()
            pltpu.sync_copy(hstage_sh, h2d_v)

            def tm(i, _):
                acc = h2d_v[pl.ds(i * L, L)]
                for r in range(1, NSUB):
                    acc = acc + h2d_v[pl.ds(r * 1024 + i * L, L)]
                gh_v[pl.ds(i * L, L)] = acc
                return 0
            lax.fori_loop(0, 1024 // L, tm, 0)
            plsc.subcore_barrier()

            def sscan(i, carry2):
                sfx_c, d_acc, nk_acc = carry2
                v = 63 - i
                h = gh_v[pl.ds(v * L, L)]
                sfx_incl = lax.rev(plsc.cumsum(lax.rev(h, (0,))), (0,))
                sfx = sfx_incl - h + sfx_c
                cond = (sfx < k_rem) & (sfx + h >= k_rem)
                tg = v * L + lane
                d_acc = d_acc + jnp.sum(jnp.where(cond, tg, 0))
                nk_acc = nk_acc + jnp.sum(jnp.where(cond, k_rem - sfx, 0))
                return sfx_c + jnp.sum(h), d_acc, nk_acc
            _, dig, newk = lax.fori_loop(
                0, 1024 // L, sscan,
                (jnp.int32(0), jnp.int32(0), jnp.int32(0)))
            return prefix | lax.shift_left(dig, shift), newk

        t_bits, need = lax.fori_loop(0, 3, one_pass,
                                     (jnp.int32(0), jnp.int32(K)))
        cnt_gt_total = K - need

      with jax.named_scope("sel_counts"):
        def csweep(i, carry2):
            cg, ce = carry2
            b = plsc.bitcast(err_v[pl.ds(i * L, L)], jnp.int32)
            tb = jnp.full((L,), t_bits, jnp.int32)
            cg = cg + jnp.sum((b > tb).astype(jnp.int32))
            ce = ce + jnp.sum((b == tb).astype(jnp.int32))
            return cg, ce
        cgt, ceq = lax.fori_loop(0, NVREG, csweep,
                                 (jnp.int32(0), jnp.int32(0)))
        cnt_v[...] = jnp.where(lane == 0, cgt,
                               jnp.where(lane == 1, ceq, 0))
        pltpu.sync_copy(cnt_v, cnt_sh.at[pl.ds(s * L, L)])
        plsc.subcore_barrier()
        pltpu.sync_copy(cnt_sh, cnt2d_v)

        def ovfred(t, o):
            row = cnt2d_v[pl.ds(t * L, L)]
            ge_t = (jnp.sum(jnp.where(lane == 0, row, 0))
                    + jnp.sum(jnp.where(lane == 1, row, 0)))
            return o | (ge_t > CAP).astype(jnp.int32)
        ovf = lax.fori_loop(0, NSUB, ovfred, jnp.int32(0))

      with jax.named_scope("sel_dsweep"):
        @pl.when(ovf == 0)
        def _fast():
            def pfill(i, _):
                wpad = N_PIX + s * CAP + i * L + lane
                werr_v[pl.ds(i * L, L)] = jnp.full((L,), -1.0, jnp.float32)
                wpos_v[pl.ds(i * L, L)] = wpad
                widx_v[pl.ds(i * L, L)] = jnp.zeros((L,), jnp.int32)
                return 0
            lax.fori_loop(0, CAP // L, pfill, 0)

            def fsweep(i, run):
                ev = err_v[pl.ds(i * L, L)]
                b = plsc.bitcast(ev, jnp.int32)
                tb = jnp.full((L,), t_bits, jnp.int32)
                m_ge = b >= tb
                r = plsc.cumsum(m_ge.astype(jnp.int32)) + run - 1
                pos = shard + i * L + lane
                plsc.store_scatter(werr_v, [r], ev, mask=m_ge)
                plsc.store_scatter(wpos_v, [r], pos, mask=m_ge)
                plsc.store_scatter(widx_v, [r], idx_v[pl.ds(i * L, L)],
                                   mask=m_ge)
                return run + jnp.sum(m_ge.astype(jnp.int32))
            lax.fori_loop(0, NVREG, fsweep, jnp.int32(0))

            pltpu.sync_copy(werr_v, serr_hbm.at[pl.ds(s * CAP, CAP)])
            pltpu.sync_copy(wpos_v, spos_hbm.at[pl.ds(s * CAP, CAP)])
            pltpu.sync_copy(widx_v, sidx_hbm.at[pl.ds(s * CAP, CAP)])

        @pl.when(ovf != 0)
        def _slow():
            def bases(t, carry2):
                gb, eb = carry2
                row = cnt2d_v[pl.ds(t * L, L)]
                take = (t < s).astype(jnp.int32)
                gb = gb + take * jnp.sum(jnp.where(lane == 0, row, 0))
                eb = eb + take * jnp.sum(jnp.where(lane == 1, row, 0))
                return gb, eb
            gt_base, eq_base = lax.fori_loop(0, NSUB, bases,
                                             (jnp.int32(0), jnp.int32(0)))

            def dsweep(i, carry2):
                rg, re = carry2
                b = plsc.bitcast(err_v[pl.ds(i * L, L)], jnp.int32)
                tb = jnp.full((L,), t_bits, jnp.int32)
                m_gt = b > tb
                m_eq = b == tb
                r_gt = plsc.cumsum(m_gt.astype(jnp.int32))
                r_eq = plsc.cumsum(m_eq.astype(jnp.int32))
                pos = shard + i * L + lane
                dgt = gt_base + rg + r_gt - 1
                der = eq_base + re + r_eq - 1
                kept = m_eq & (der < need)
                dest = jnp.where(m_gt, dgt,
                                 jnp.where(kept, cnt_gt_total + der, W + pos))
                dest_v[pl.ds(i * L, L)] = dest
                buf_v[pl.ds(i * L, L)] = pos
                return (rg + jnp.sum(m_gt.astype(jnp.int32)),
                        re + jnp.sum(m_eq.astype(jnp.int32)))
            lax.fori_loop(0, NVREG, dsweep, (jnp.int32(0), jnp.int32(0)))

            pltpu.async_copy(err_v, serr_hbm.at[dest_v], sem).wait()
            pltpu.async_copy(buf_v, spos_hbm.at[dest_v], sem).wait()
            pltpu.async_copy(idx_v, sidx_hbm.at[dest_v], sem).wait()

            def pfill2(i, _):
                werr_v[pl.ds(i * L, L)] = jnp.full((L,), -1.0, jnp.float32)
                wpos_v[pl.ds(i * L, L)] = N_PIX + s * TRK + i * L + lane
                widx_v[pl.ds(i * L, L)] = jnp.zeros((L,), jnp.int32)
                return 0
            lax.fori_loop(0, TRK // L, pfill2, 0)
            pltpu.sync_copy(werr_v.at[pl.ds(0, TRK)],
                            serr_hbm.at[pl.ds(K + s * TRK, TRK)])
            pltpu.sync_copy(wpos_v.at[pl.ds(0, TRK)],
                            spos_hbm.at[pl.ds(K + s * TRK, TRK)])
            pltpu.sync_copy(widx_v.at[pl.ds(0, TRK)],
                            sidx_hbm.at[pl.ds(K + s * TRK, TRK)])


def _sc_call(errors, indices, old_errors, old_indices):
    mesh = plsc.VectorSubcoreMesh(core_axis_name="c", subcore_axis_name="s",
                                  num_cores=2, num_subcores=NSUB)
    f = pl.kernel(
        _sc_body,
        mesh=mesh,
        out_type=(
            jax.ShapeDtypeStruct((K,), jnp.float32),
            jax.ShapeDtypeStruct((OUT_PAD,), jnp.float32),
            jax.ShapeDtypeStruct((OUT_PAD,), jnp.int32),
            jax.ShapeDtypeStruct((OUT_PAD,), jnp.int32),
        ),
        compiler_params=pltpu.CompilerParams(needs_layout_passes=False),
        scratch_types=[
            pltpu.VMEM_SHARED((N_PIX,), jnp.int32),
            pltpu.VMEM_SHARED((NSUB * K,), jnp.float32),
            pltpu.VMEM_SHARED((K,), jnp.float32),
            pltpu.VMEM_SHARED((NSUB * 1024,), jnp.int32),
            pltpu.VMEM_SHARED((NSUB * L,), jnp.int32),
            pltpu.VMEM((EPT,), jnp.int32),
            pltpu.VMEM((EPT,), jnp.float32),
            pltpu.VMEM((EPT,), jnp.int32),
            pltpu.VMEM((EPT,), jnp.int32),
            pltpu.VMEM((NSUB * 1024,), jnp.int32),
            pltpu.VMEM((1024,), jnp.int32),
            pltpu.VMEM((1024,), jnp.int32),
            pltpu.VMEM((K,), jnp.float32),
            pltpu.VMEM((TRK,), jnp.float32),
            pltpu.VMEM((TRK,), jnp.int32),
            pltpu.VMEM((TRK,), jnp.int32),
            pltpu.VMEM((TRK,), jnp.float32),
            pltpu.VMEM((L,), jnp.int32),
            pltpu.VMEM((NSUB * L,), jnp.int32),
            pltpu.VMEM((CAP,), jnp.float32),
            pltpu.VMEM((CAP,), jnp.int32),
            pltpu.VMEM((CAP,), jnp.int32),
            pltpu.SemaphoreType.DMA,
        ],
    )
    return f(errors, indices, old_errors, old_indices)


def _lin_iota(nr):
    r = lax.broadcasted_iota(jnp.int32, (nr, C), 0)
    c = lax.broadcasted_iota(jnp.int32, (nr, C), 1)
    return r * C + c


def _xor_shuffle(x, d, nr):
    if d < C:
        fwd = pltpu.roll(x, C - d, 1)
        bwd = pltpu.roll(x, d, 1)
        sel = (lax.broadcasted_iota(jnp.int32, (nr, C), 1) & d) == 0
    else:
        dr = d // C
        fwd = pltpu.roll(x, nr - dr, 0)
        bwd = pltpu.roll(x, dr, 0)
        sel = (lax.broadcasted_iota(jnp.int32, (nr, C), 0) & dr) == 0
    return jnp.where(sel, fwd, bwd)


def _bitonic3(key, pos, idx, nr, levels):
    e = _lin_iota(nr)
    for k in range(1, levels + 1):
        for j in range(k - 1, -1, -1):
            d = 1 << j
            kk = _xor_shuffle(key, d, nr)
            pp = _xor_shuffle(pos, d, nr)
            ii = _xor_shuffle(idx, d, nr)
            upper = (e & d) != 0
            if k < levels:
                asc = (e & (1 << k)) != 0
            else:
                asc = jnp.zeros_like(upper)
            keep_larger = ~(upper ^ asc)
            mine_gt = (key > kk) | ((key == kk) & (pos < pp))
            take_mine = ~(keep_larger ^ mine_gt)
            key = jnp.where(take_mine, key, kk)
            pos = jnp.where(take_mine, pos, pp)
            idx = jnp.where(take_mine, idx, ii)
    return key, pos, idx


def _bitonic1(v, nr, levels):
    e = _lin_iota(nr)
    for k in range(1, levels + 1):
        for j in range(k - 1, -1, -1):
            d = 1 << j
            vv = _xor_shuffle(v, d, nr)
            upper = (e & d) != 0
            if k < levels:
                asc = (e & (1 << k)) != 0
            else:
                asc = jnp.zeros_like(upper)
            keep_larger = ~(upper ^ asc)
            v = jnp.where(keep_larger, jnp.maximum(v, vv), jnp.minimum(v, vv))
    return v


def _sort_merge_body(se_ref, sp_ref, si_ref, upd_ref, oi_ref, oe_ref, ooi_ref):
    key, pos, idx = _bitonic3(se_ref[...], sp_ref[...], si_ref[...], R2, 14)
    key = key[:R, :]
    idx = idx[:R, :]
    me = _bitonic1(upd_ref[...], R, 13)
    surpassed = key > me
    oe_ref[...] = jnp.where(surpassed, key, me)
    ooi_ref[...] = jnp.where(surpassed, idx, oi_ref[...])


def _tc_call(sel_err, sel_pos, sel_idx, updated, old_idx):
    return pl.pallas_call(
        _sort_merge_body,
        out_shape=(
            jax.ShapeDtypeStruct((R, C), jnp.float32),
            jax.ShapeDtypeStruct((R, C), jnp.int32),
        ),
    )(sel_err, sel_pos, sel_idx, updated, old_idx)


def kernel(errors, indices, old_errors, old_indices):
    errors_flat = errors.reshape(-1)
    indices_flat = indices.reshape(-1)
    upd, sel_err, sel_pos, sel_idx = _sc_call(
        errors_flat, indices_flat, old_errors, old_indices)
    oe, oi = _tc_call(
        sel_err[:W].reshape(R2, C), sel_pos[:W].reshape(R2, C),
        sel_idx[:W].reshape(R2, C), upd.reshape(R, C),
        old_indices.reshape(R, C))
    return oe.reshape(K), oi.reshape(K)


# speedup vs baseline: 22.5559x; 1.0205x over previous
import jax
import jax.numpy as jnp
from jax import lax
from jax.experimental import pallas as pl
from jax.experimental.pallas import tpu as pltpu
from jax.experimental.pallas import tpu_sc as plsc

N_PIX = 262144
K = 8192
NSUB = 16
L = 16
EPT = N_PIX // NSUB
TRK = K // NSUB
NVREG = EPT // L
CAP = 1024
W = NSUB * CAP
OUT_PAD = W + N_PIX
R, C = 64, 128
R2 = 128


def _sc_body(err_hbm, idx_hbm, oerr_hbm, oidx_hbm,
             upd_hbm, serr_hbm, spos_hbm, sidx_hbm,
             s_sh, stage_sh, gmax_sh, hstage_sh, cnt_sh,
             idx_v, err_v, buf_v, dest_v, h2d_v, h1d_v, gh_v,
             smax_v, out_v, oi_v, oslot_v, oerr_v, cnt_v,
             cnt2d_v, werr_v, wpos_v, widx_v, sem):
    c = lax.axis_index("c")
    s = lax.axis_index("s")
    lane = lax.iota(jnp.int32, L)
    shard = s * EPT

    pltpu.sync_copy(idx_hbm.at[pl.ds(shard, EPT)], idx_v)
    pltpu.sync_copy(err_hbm.at[pl.ds(shard, EPT)], err_v)

    @pl.when(c == 0)
    def _join():
      with jax.named_scope("join_setup"):
        def fneg(i, _):
            dest_v[pl.ds(i * L, L)] = jnp.full((L,), -1, jnp.int32)
            return 0
        lax.fori_loop(0, EPT // L, fneg, 0)
        pltpu.sync_copy(dest_v, s_sh.at[pl.ds(s * EPT, EPT)])
        plsc.subcore_barrier()

        pltpu.sync_copy(oidx_hbm.at[pl.ds(s * TRK, TRK)], oi_v)

        def fslot(i, _):
            oslot_v[pl.ds(i * L, L)] = s * TRK + i * L + lane
            return 0
        lax.fori_loop(0, TRK // L, fslot, 0)
        pltpu.async_copy(oslot_v, s_sh.at[oi_v], sem).wait()
        plsc.subcore_barrier()

        pltpu.async_copy(s_sh.at[idx_v], buf_v, sem).wait()

        def fzero(i, _):
            smax_v[pl.ds(i * L, L)] = jnp.zeros((L,), jnp.float32)
            return 0
        lax.fori_loop(0, K // L, fzero, 0)

      with jax.named_scope("join_accum"):
        def hsweep(i, run):
            sl = buf_v[pl.ds(i * L, L)]
            hit = sl >= 0
            r = plsc.cumsum(hit.astype(jnp.int32)) + run - 1
            plsc.store_scatter(dest_v, [r], sl, mask=hit)
            plsc.store_scatter(h2d_v, [r], i * L + lane, mask=hit)
            return run + jnp.sum(hit.astype(jnp.int32))
        nh = lax.fori_loop(0, NVREG, hsweep, jnp.int32(0))

        def rmw(i, _):
            sl = dest_v[pl.ds(i * L, L)]
            lp = h2d_v[pl.ds(i * L, L)]
            vm = (i * L + lane) < nh
            ks = jnp.where(vm, sl, 0)
            ev = plsc.load_gather(err_v, [jnp.where(vm, lp, 0)], mask=vm)

            def cond(carry):
                return jnp.any(carry[0])

            def body(carry):
                rem, it = carry
                cur = plsc.load_gather(smax_v, [ks], mask=rem)
                newv = jnp.maximum(cur, ev)
                plsc.store_scatter(smax_v, [ks], newv, mask=rem)
                back = plsc.load_gather(smax_v, [ks], mask=rem)
                return rem & (back < newv), it + jnp.int32(1)

            lax.while_loop(cond, body, (vm, jnp.int32(0)))
            return 0
        lax.fori_loop(0, (nh + L - 1) // L, rmw, 0)

      with jax.named_scope("join_merge"):
        plsc.subcore_barrier()
        pltpu.sync_copy(smax_v, stage_sh.at[pl.ds(s * K, K)])
        plsc.subcore_barrier()
        pltpu.sync_copy(stage_sh.at[pl.ds(s * TRK, TRK)], out_v)

        def rmerge(r, _):
            pltpu.sync_copy(stage_sh.at[pl.ds(r * K + s * TRK, TRK)], oerr_v)

            def red(i, _):
                out_v[pl.ds(i * L, L)] = jnp.maximum(
                    out_v[pl.ds(i * L, L)], oerr_v[pl.ds(i * L, L)])
                return 0
            lax.fori_loop(0, TRK // L, red, 0)
            return 0
        lax.fori_loop(1, NSUB, rmerge, 0)
        pltpu.sync_copy(out_v, gmax_sh.at[pl.ds(s * TRK, TRK)])
        plsc.subcore_barrier()

      with jax.named_scope("join_answer"):
        pltpu.sync_copy(gmax_sh, smax_v)
        pltpu.sync_copy(oerr_hbm.at[pl.ds(s * TRK, TRK)], oerr_v)
        pltpu.async_copy(s_sh.at[oi_v], oslot_v, sem).wait()

        def answer(i, _):
            sl = oslot_v[pl.ds(i * L, L)]
            g = plsc.load_gather(smax_v, [sl])
            out_v[pl.ds(i * L, L)] = jnp.maximum(oerr_v[pl.ds(i * L, L)], g)
            return 0
        lax.fori_loop(0, TRK // L, answer, 0)
        pltpu.sync_copy(out_v, upd_hbm.at[pl.ds(s * TRK, TRK)])

    @pl.when(c == 1)
    def _select():
      with jax.named_scope("sel_passes"):
        def one_pass(p, carry):
            prefix, k_rem = carry
            shift = 20 - 10 * p

            def hz(i, _):
                for r in range(NSUB):
                    h2d_v[pl.ds(r * 1024 + i * L, L)] = jnp.zeros((L,), jnp.int32)
                return 0
            lax.fori_loop(0, 1024 // L, hz, 0)

            def scan(i, _):
                shv = jnp.full((L,), shift, jnp.int32)
                pref = jnp.full((L,), lax.shift_right_logical(
                    prefix, shift + 10), jnp.int32)
                ones = jnp.full((L,), 1, jnp.int32)
                for u in range(4):
                    b = plsc.bitcast(
                        err_v[pl.ds((i * 4 + u) * L, L)], jnp.int32)
                    d = jnp.bitwise_and(lax.shift_right_logical(b, shv),
                                        jnp.full((L,), 1023, jnp.int32))
                    hi = lax.shift_right_logical(b, shv + 10)
                    plsc.addupdate_scatter(h2d_v, [lane * 1024 + d],
                                           ones, mask=hi == pref)
                return 0
            lax.fori_loop(0, NVREG // 4, scan, 0)

            def lm(i, _):
                acc = h2d_v[pl.ds(i * L, L)]
                for r in range(1, NSUB):
                    acc = acc + h2d_v[pl.ds(r * 1024 + i * L, L)]
                h1d_v[pl.ds(i * L, L)] = acc
                return 0
            lax.fori_loop(0, 1024 // L, lm, 0)

            pltpu.sync_copy(h1d_v, hstage_sh.at[pl.ds(s * 1024, 1024)])
            plsc.subcore_barrier()
            pltpu.sync_copy(hstage_sh, h2d_v)

            def tm(i, _):
                acc = h2d_v[pl.ds(i * L, L)]
                for r in range(1, NSUB):
                    acc = acc + h2d_v[pl.ds(r * 1024 + i * L, L)]
                gh_v[pl.ds(i * L, L)] = acc
                return 0
            lax.fori_loop(0, 1024 // L, tm, 0)
            plsc.subcore_barrier()

            def sscan(i, carry2):
                sfx_c, d_acc, nk_acc = carry2
                v = 63 - i
                h = gh_v[pl.ds(v * L, L)]
                sfx_incl = lax.rev(plsc.cumsum(lax.rev(h, (0,))), (0,))
                sfx = sfx_incl - h + sfx_c
                cond = (sfx < k_rem) & (sfx + h >= k_rem)
                tg = v * L + lane
                d_acc = d_acc + jnp.sum(jnp.where(cond, tg, 0))
                nk_acc = nk_acc + jnp.sum(jnp.where(cond, k_rem - sfx, 0))
                return sfx_c + jnp.sum(h), d_acc, nk_acc
            _, dig, newk = lax.fori_loop(
                0, 1024 // L, sscan,
                (jnp.int32(0), jnp.int32(0), jnp.int32(0)))
            return prefix | lax.shift_left(dig, shift), newk

        t_bits, need = lax.fori_loop(0, 3, one_pass,
                                     (jnp.int32(0), jnp.int32(K)))
        cnt_gt_total = K - need

      with jax.named_scope("sel_counts"):
        def csweep(i, carry2):
            cg, ce = carry2
            b = plsc.bitcast(err_v[pl.ds(i * L, L)], jnp.int32)
            tb = jnp.full((L,), t_bits, jnp.int32)
            cg = cg + jnp.sum((b > tb).astype(jnp.int32))
            ce = ce + jnp.sum((b == tb).astype(jnp.int32))
            return cg, ce
        cgt, ceq = lax.fori_loop(0, NVREG, csweep,
                                 (jnp.int32(0), jnp.int32(0)))
        cnt_v[...] = jnp.where(lane == 0, cgt,
                               jnp.where(lane == 1, ceq, 0))
        pltpu.sync_copy(cnt_v, cnt_sh.at[pl.ds(s * L, L)])
        plsc.subcore_barrier()
        pltpu.sync_copy(cnt_sh, cnt2d_v)

        def ovfred(t, o):
            row = cnt2d_v[pl.ds(t * L, L)]
            ge_t = (jnp.sum(jnp.where(lane == 0, row, 0))
                    + jnp.sum(jnp.where(lane == 1, row, 0)))
            return o | (ge_t > CAP).astype(jnp.int32)
        ovf = lax.fori_loop(0, NSUB, ovfred, jnp.int32(0))

      with jax.named_scope("sel_dsweep"):
        @pl.when(ovf == 0)
        def _fast():
            def pfill(i, _):
                wpad = N_PIX + s * CAP + i * L + lane
                werr_v[pl.ds(i * L, L)] = jnp.full((L,), -1.0, jnp.float32)
                wpos_v[pl.ds(i * L, L)] = wpad
                widx_v[pl.ds(i * L, L)] = jnp.zeros((L,), jnp.int32)
                return 0
            lax.fori_loop(0, CAP // L, pfill, 0)

            def fsweep(i, run):
                ev = err_v[pl.ds(i * L, L)]
                b = plsc.bitcast(ev, jnp.int32)
                tb = jnp.full((L,), t_bits, jnp.int32)
                m_ge = b >= tb
                r = plsc.cumsum(m_ge.astype(jnp.int32)) + run - 1
                pos = shard + i * L + lane
                plsc.store_scatter(werr_v, [r], ev, mask=m_ge)
                plsc.store_scatter(wpos_v, [r], pos, mask=m_ge)
                plsc.store_scatter(widx_v, [r], idx_v[pl.ds(i * L, L)],
                                   mask=m_ge)
                return run + jnp.sum(m_ge.astype(jnp.int32))
            lax.fori_loop(0, NVREG, fsweep, jnp.int32(0))

            pltpu.sync_copy(werr_v, serr_hbm.at[pl.ds(s * CAP, CAP)])
            pltpu.sync_copy(wpos_v, spos_hbm.at[pl.ds(s * CAP, CAP)])
            pltpu.sync_copy(widx_v, sidx_hbm.at[pl.ds(s * CAP, CAP)])

        @pl.when(ovf != 0)
        def _slow():
            def bases(t, carry2):
                gb, eb = carry2
                row = cnt2d_v[pl.ds(t * L, L)]
                take = (t < s).astype(jnp.int32)
                gb = gb + take * jnp.sum(jnp.where(lane == 0, row, 0))
                eb = eb + take * jnp.sum(jnp.where(lane == 1, row, 0))
                return gb, eb
            gt_base, eq_base = lax.fori_loop(0, NSUB, bases,
                                             (jnp.int32(0), jnp.int32(0)))

            def dsweep(i, carry2):
                rg, re = carry2
                b = plsc.bitcast(err_v[pl.ds(i * L, L)], jnp.int32)
                tb = jnp.full((L,), t_bits, jnp.int32)
                m_gt = b > tb
                m_eq = b == tb
                r_gt = plsc.cumsum(m_gt.astype(jnp.int32))
                r_eq = plsc.cumsum(m_eq.astype(jnp.int32))
                pos = shard + i * L + lane
                dgt = gt_base + rg + r_gt - 1
                der = eq_base + re + r_eq - 1
                kept = m_eq & (der < need)
                dest = jnp.where(m_gt, dgt,
                                 jnp.where(kept, cnt_gt_total + der, W + pos))
                dest_v[pl.ds(i * L, L)] = dest
                buf_v[pl.ds(i * L, L)] = pos
                return (rg + jnp.sum(m_gt.astype(jnp.int32)),
                        re + jnp.sum(m_eq.astype(jnp.int32)))
            lax.fori_loop(0, NVREG, dsweep, (jnp.int32(0), jnp.int32(0)))

            pltpu.async_copy(err_v, serr_hbm.at[dest_v], sem).wait()
            pltpu.async_copy(buf_v, spos_hbm.at[dest_v], sem).wait()
            pltpu.async_copy(idx_v, sidx_hbm.at[dest_v], sem).wait()

            def pfill2(i, _):
                werr_v[pl.ds(i * L, L)] = jnp.full((L,), -1.0, jnp.float32)
                wpos_v[pl.ds(i * L, L)] = N_PIX + s * TRK + i * L + lane
                widx_v[pl.ds(i * L, L)] = jnp.zeros((L,), jnp.int32)
                return 0
            lax.fori_loop(0, TRK // L, pfill2, 0)
            pltpu.sync_copy(werr_v.at[pl.ds(0, TRK)],
                            serr_hbm.at[pl.ds(K + s * TRK, TRK)])
            pltpu.sync_copy(wpos_v.at[pl.ds(0, TRK)],
                            spos_hbm.at[pl.ds(K + s * TRK, TRK)])
            pltpu.sync_copy(widx_v.at[pl.ds(0, TRK)],
                            sidx_hbm.at[pl.ds(K + s * TRK, TRK)])


def _sc_call(errors, indices, old_errors, old_indices):
    mesh = plsc.VectorSubcoreMesh(core_axis_name="c", subcore_axis_name="s",
                                  num_cores=2, num_subcores=NSUB)
    f = pl.kernel(
        _sc_body,
        mesh=mesh,
        out_type=(
            jax.ShapeDtypeStruct((K,), jnp.float32),
            jax.ShapeDtypeStruct((OUT_PAD,), jnp.float32),
            jax.ShapeDtypeStruct((OUT_PAD,), jnp.int32),
            jax.ShapeDtypeStruct((OUT_PAD,), jnp.int32),
        ),
        compiler_params=pltpu.CompilerParams(needs_layout_passes=False),
        scratch_types=[
            pltpu.VMEM_SHARED((N_PIX,), jnp.int32),
            pltpu.VMEM_SHARED((NSUB * K,), jnp.float32),
            pltpu.VMEM_SHARED((K,), jnp.float32),
            pltpu.VMEM_SHARED((NSUB * 1024,), jnp.int32),
            pltpu.VMEM_SHARED((NSUB * L,), jnp.int32),
            pltpu.VMEM((EPT,), jnp.int32),
            pltpu.VMEM((EPT,), jnp.float32),
            pltpu.VMEM((EPT,), jnp.int32),
            pltpu.VMEM((EPT,), jnp.int32),
            pltpu.VMEM((NSUB * 1024,), jnp.int32),
            pltpu.VMEM((1024,), jnp.int32),
            pltpu.VMEM((1024,), jnp.int32),
            pltpu.VMEM((K,), jnp.float32),
            pltpu.VMEM((TRK,), jnp.float32),
            pltpu.VMEM((TRK,), jnp.int32),
            pltpu.VMEM((TRK,), jnp.int32),
            pltpu.VMEM((TRK,), jnp.float32),
            pltpu.VMEM((L,), jnp.int32),
            pltpu.VMEM((NSUB * L,), jnp.int32),
            pltpu.VMEM((CAP,), jnp.float32),
            pltpu.VMEM((CAP,), jnp.int32),
            pltpu.VMEM((CAP,), jnp.int32),
            pltpu.SemaphoreType.DMA,
        ],
    )
    return f(errors, indices, old_errors, old_indices)


def _lin_iota(nr):
    r = lax.broadcasted_iota(jnp.int32, (nr, C), 0)
    c = lax.broadcasted_iota(jnp.int32, (nr, C), 1)
    return r * C + c


def _xor_shuffle(x, d, nr):
    if d < C:
        fwd = pltpu.roll(x, C - d, 1)
        bwd = pltpu.roll(x, d, 1)
        sel = (lax.broadcasted_iota(jnp.int32, (nr, C), 1) & d) == 0
    else:
        dr = d // C
        fwd = pltpu.roll(x, nr - dr, 0)
        bwd = pltpu.roll(x, dr, 0)
        sel = (lax.broadcasted_iota(jnp.int32, (nr, C), 0) & dr) == 0
    return jnp.where(sel, fwd, bwd)


def _bitonic3(key, pos, idx, nr, levels):
    e = _lin_iota(nr)
    for k in range(1, levels + 1):
        for j in range(k - 1, -1, -1):
            d = 1 << j
            kk = _xor_shuffle(key, d, nr)
            pp = _xor_shuffle(pos, d, nr)
            ii = _xor_shuffle(idx, d, nr)
            upper = (e & d) != 0
            if k < levels:
                asc = (e & (1 << k)) != 0
            else:
                asc = jnp.zeros_like(upper)
            keep_larger = ~(upper ^ asc)
            mine_gt = (key > kk) | ((key == kk) & (pos < pp))
            take_mine = ~(keep_larger ^ mine_gt)
            key = jnp.where(take_mine, key, kk)
            pos = jnp.where(take_mine, pos, pp)
            idx = jnp.where(take_mine, idx, ii)
    return key, pos, idx


def _bitonic1(v, nr, levels):
    e = _lin_iota(nr)
    for k in range(1, levels + 1):
        for j in range(k - 1, -1, -1):
            d = 1 << j
            vv = _xor_shuffle(v, d, nr)
            upper = (e & d) != 0
            if k < levels:
                asc = (e & (1 << k)) != 0
            else:
                asc = jnp.zeros_like(upper)
            keep_larger = ~(upper ^ asc)
            v = jnp.where(keep_larger, jnp.maximum(v, vv), jnp.minimum(v, vv))
    return v


def _sort_merge_body(se_ref, sp_ref, si_ref, upd_ref, oi_ref, oe_ref, ooi_ref):
    key, pos, idx = _bitonic3(se_ref[...], sp_ref[...], si_ref[...], R2, 14)
    key = key[:R, :]
    idx = idx[:R, :]
    me = _bitonic1(upd_ref[...], R, 13)
    surpassed = key > me
    oe_ref[...] = jnp.where(surpassed, key, me)
    ooi_ref[...] = jnp.where(surpassed, idx, oi_ref[...])


def _tc_call(sel_err, sel_pos, sel_idx, updated, old_idx):
    return pl.pallas_call(
        _sort_merge_body,
        out_shape=(
            jax.ShapeDtypeStruct((R, C), jnp.float32),
            jax.ShapeDtypeStruct((R, C), jnp.int32),
        ),
    )(sel_err, sel_pos, sel_idx, updated, old_idx)


def kernel(errors, indices, old_errors, old_indices):
    errors_flat = errors.reshape(-1)
    indices_flat = indices.reshape(-1)
    upd, sel_err, sel_pos, sel_idx = _sc_call(
        errors_flat, indices_flat, old_errors, old_indices)
    oe, oi = _tc_call(
        sel_err[:W].reshape(R2, C), sel_pos[:W].reshape(R2, C),
        sel_idx[:W].reshape(R2, C), upd.reshape(R, C),
        old_indices.reshape(R, C))
    return oe.reshape(K), oi.reshape(K)


# speedup vs baseline: 25.2922x; 1.1213x over previous
import jax
import jax.numpy as jnp
from jax import lax
from jax.experimental import pallas as pl
from jax.experimental.pallas import tpu as pltpu
from jax.experimental.pallas import tpu_sc as plsc

N_PIX = 262144
K = 8192
NSUB = 16
L = 16
EPT = N_PIX // NSUB
TRK = K // NSUB
NVREG = EPT // L
CAP = 1024
W = NSUB * CAP
OUT_PAD = W + N_PIX
R, C = 64, 128
R2 = 128


def _sc_body(err_hbm, idx_hbm, oerr_hbm, oidx_hbm,
             upd_hbm, serr_hbm, spos_hbm, sidx_hbm,
             s_sh, stage_sh, gmax_sh, hstage_sh, cnt_sh,
             idx_v, err_v, buf_v, dest_v, h2d_v, h1d_v, gh_v,
             smax_v, out_v, oi_v, oslot_v, oerr_v, cnt_v,
             cnt2d_v, werr_v, wpos_v, widx_v, sem):
    c = lax.axis_index("c")
    s = lax.axis_index("s")
    lane = lax.iota(jnp.int32, L)
    shard = s * EPT

    pltpu.sync_copy(idx_hbm.at[pl.ds(shard, EPT)], idx_v)
    pltpu.sync_copy(err_hbm.at[pl.ds(shard, EPT)], err_v)

    @pl.when(c == 0)
    def _join():
      with jax.named_scope("join_setup"):
        def fneg(i, _):
            dest_v[pl.ds(i * L, L)] = jnp.full((L,), -1, jnp.int32)
            return 0
        lax.fori_loop(0, EPT // L, fneg, 0)
        pltpu.sync_copy(dest_v, s_sh.at[pl.ds(s * EPT, EPT)])
        plsc.subcore_barrier()

        pltpu.sync_copy(oidx_hbm.at[pl.ds(s * TRK, TRK)], oi_v)

        def fslot(i, _):
            oslot_v[pl.ds(i * L, L)] = s * TRK + i * L + lane
            return 0
        lax.fori_loop(0, TRK // L, fslot, 0)
        pltpu.async_copy(oslot_v, s_sh.at[oi_v], sem).wait()
        plsc.subcore_barrier()

        pltpu.async_copy(s_sh.at[idx_v], buf_v, sem).wait()

        def fzero(i, _):
            smax_v[pl.ds(i * L, L)] = jnp.zeros((L,), jnp.float32)
            return 0
        lax.fori_loop(0, K // L, fzero, 0)

      with jax.named_scope("join_accum"):
        def hsweep(i, run):
            sl = buf_v[pl.ds(i * L, L)]
            hit = sl >= 0
            r = plsc.cumsum(hit.astype(jnp.int32)) + run - 1
            plsc.store_scatter(dest_v, [r], sl, mask=hit)
            plsc.store_scatter(h2d_v, [r], i * L + lane, mask=hit)
            return run + jnp.sum(hit.astype(jnp.int32))
        nh = lax.fori_loop(0, NVREG, hsweep, jnp.int32(0))

        def rmw(i, _):
            sl = dest_v[pl.ds(i * L, L)]
            lp = h2d_v[pl.ds(i * L, L)]
            vm = (i * L + lane) < nh
            ks = jnp.where(vm, sl, 0)
            ev = plsc.load_gather(err_v, [jnp.where(vm, lp, 0)], mask=vm)

            def cond(carry):
                return jnp.any(carry[0])

            def body(carry):
                rem, it = carry
                cur = plsc.load_gather(smax_v, [ks], mask=rem)
                newv = jnp.maximum(cur, ev)
                plsc.store_scatter(smax_v, [ks], newv, mask=rem)
                back = plsc.load_gather(smax_v, [ks], mask=rem)
                return rem & (back < newv), it + jnp.int32(1)

            lax.while_loop(cond, body, (vm, jnp.int32(0)))
            return 0
        lax.fori_loop(0, (nh + L - 1) // L, rmw, 0)

      with jax.named_scope("join_merge"):
        plsc.subcore_barrier()
        pltpu.sync_copy(smax_v, stage_sh.at[pl.ds(s * K, K)])
        plsc.subcore_barrier()
        pltpu.sync_copy(stage_sh.at[pl.ds(s * TRK, TRK)], out_v)

        def rmerge(r, _):
            pltpu.sync_copy(stage_sh.at[pl.ds(r * K + s * TRK, TRK)], oerr_v)

            def red(i, _):
                out_v[pl.ds(i * L, L)] = jnp.maximum(
                    out_v[pl.ds(i * L, L)], oerr_v[pl.ds(i * L, L)])
                return 0
            lax.fori_loop(0, TRK // L, red, 0)
            return 0
        lax.fori_loop(1, NSUB, rmerge, 0)
        pltpu.sync_copy(out_v, gmax_sh.at[pl.ds(s * TRK, TRK)])
        plsc.subcore_barrier()

      with jax.named_scope("join_answer"):
        pltpu.sync_copy(gmax_sh, smax_v)
        pltpu.sync_copy(oerr_hbm.at[pl.ds(s * TRK, TRK)], oerr_v)
        pltpu.async_copy(s_sh.at[oi_v], oslot_v, sem).wait()

        def answer(i, _):
            sl = oslot_v[pl.ds(i * L, L)]
            g = plsc.load_gather(smax_v, [sl])
            out_v[pl.ds(i * L, L)] = jnp.maximum(oerr_v[pl.ds(i * L, L)], g)
            return 0
        lax.fori_loop(0, TRK // L, answer, 0)
        pltpu.sync_copy(out_v, upd_hbm.at[pl.ds(s * TRK, TRK)])

    @pl.when(c == 1)
    def _select():
      with jax.named_scope("sel_passes"):
        def one_pass(p, carry):
            prefix, k_rem = carry
            shift = 20 - 10 * p

            def hz(i, _):
                for r in range(NSUB):
                    h2d_v[pl.ds(r * 1024 + i * L, L)] = jnp.zeros((L,), jnp.int32)
                return 0
            lax.fori_loop(0, 1024 // L, hz, 0)

            def scan(i, _):
                shv = jnp.full((L,), shift, jnp.int32)
                pref = jnp.full((L,), lax.shift_right_logical(
                    prefix, shift + 10), jnp.int32)
                ones = jnp.full((L,), 1, jnp.int32)
                for u in range(4):
                    b = plsc.bitcast(
                        err_v[pl.ds((i * 4 + u) * L, L)], jnp.int32)
                    d = jnp.bitwise_and(lax.shift_right_logical(b, shv),
                                        jnp.full((L,), 1023, jnp.int32))
                    hi = lax.shift_right_logical(b, shv + 10)
                    plsc.addupdate_scatter(h2d_v, [lane * 1024 + d],
                                           ones, mask=hi == pref)
                return 0
            lax.fori_loop(0, NVREG // 4, scan, 0)

            def lm(i, _):
                acc = h2d_v[pl.ds(i * L, L)]
                for r in range(1, NSUB):
                    acc = acc + h2d_v[pl.ds(r * 1024 + i * L, L)]
                h1d_v[pl.ds(i * L, L)] = acc
                return 0
            lax.fori_loop(0, 1024 // L, lm, 0)

            pltpu.sync_copy(h1d_v, hstage_sh.at[pl.ds(s * 1024, 1024)])
            plsc.subcore_barrier()
            pltpu.sync_copy(hstage_sh, h2d_v)

            def tm(i, _):
                acc = h2d_v[pl.ds(i * L, L)]
                for r in range(1, NSUB):
                    acc = acc + h2d_v[pl.ds(r * 1024 + i * L, L)]
                gh_v[pl.ds(i * L, L)] = acc
                return 0
            lax.fori_loop(0, 1024 // L, tm, 0)
            plsc.subcore_barrier()

            def sscan(i, carry2):
                sfx_c, d_acc, nk_acc = carry2
                v = 63 - i
                h = gh_v[pl.ds(v * L, L)]
                sfx_incl = lax.rev(plsc.cumsum(lax.rev(h, (0,))), (0,))
                sfx = sfx_incl - h + sfx_c
                cond = (sfx < k_rem) & (sfx + h >= k_rem)
                tg = v * L + lane
                d_acc = d_acc + jnp.sum(jnp.where(cond, tg, 0))
                nk_acc = nk_acc + jnp.sum(jnp.where(cond, k_rem - sfx, 0))
                return sfx_c + jnp.sum(h), d_acc, nk_acc
            _, dig, newk = lax.fori_loop(
                0, 1024 // L, sscan,
                (jnp.int32(0), jnp.int32(0), jnp.int32(0)))
            return prefix | lax.shift_left(dig, shift), newk

        pfx2, k2 = lax.fori_loop(0, 2, one_pass,
                                 (jnp.int32(0), jnp.int32(K)))

      with jax.named_scope("sel_counts"):
        def csweep(i, cg):
            b = plsc.bitcast(err_v[pl.ds(i * L, L)], jnp.int32)
            return cg + jnp.sum((b >= jnp.full((L,), pfx2, jnp.int32))
                                .astype(jnp.int32))
        cge = lax.fori_loop(0, NVREG, csweep, jnp.int32(0))
        cnt_v[...] = jnp.where(lane == 0, cge, 0)
        pltpu.sync_copy(cnt_v, cnt_sh.at[pl.ds(s * L, L)])
        plsc.subcore_barrier()
        pltpu.sync_copy(cnt_sh, cnt2d_v)

        def ovfred(t, o):
            row = cnt2d_v[pl.ds(t * L, L)]
            ge_t = jnp.sum(jnp.where(lane == 0, row, 0))
            return o | (ge_t > CAP).astype(jnp.int32)
        ovf = lax.fori_loop(0, NSUB, ovfred, jnp.int32(0))

      with jax.named_scope("sel_dsweep"):
        @pl.when(ovf == 0)
        def _fast():
            def pfill(i, _):
                wpad = N_PIX + s * CAP + i * L + lane
                werr_v[pl.ds(i * L, L)] = jnp.full((L,), -1.0, jnp.float32)
                wpos_v[pl.ds(i * L, L)] = wpad
                widx_v[pl.ds(i * L, L)] = jnp.zeros((L,), jnp.int32)
                return 0
            lax.fori_loop(0, CAP // L, pfill, 0)

            def fsweep(i, run):
                tb = jnp.full((L,), pfx2, jnp.int32)
                for u in range(2):
                    ii = i * 2 + u
                    ev = err_v[pl.ds(ii * L, L)]
                    b = plsc.bitcast(ev, jnp.int32)
                    m_ge = b >= tb
                    r = plsc.cumsum(m_ge.astype(jnp.int32)) + run - 1
                    pos = shard + ii * L + lane
                    plsc.store_scatter(werr_v, [r], ev, mask=m_ge)
                    plsc.store_scatter(wpos_v, [r], pos, mask=m_ge)
                    plsc.store_scatter(widx_v, [r], idx_v[pl.ds(ii * L, L)],
                                       mask=m_ge)
                    run = run + jnp.sum(m_ge.astype(jnp.int32))
                return run
            lax.fori_loop(0, NVREG // 2, fsweep, jnp.int32(0))

            pltpu.sync_copy(werr_v, serr_hbm.at[pl.ds(s * CAP, CAP)])
            pltpu.sync_copy(wpos_v, spos_hbm.at[pl.ds(s * CAP, CAP)])
            pltpu.sync_copy(widx_v, sidx_hbm.at[pl.ds(s * CAP, CAP)])

        @pl.when(ovf != 0)
        def _slow():
            t_bits, need = one_pass(jnp.int32(2), (pfx2, k2))
            cnt_gt_total = K - need

            def csweep2(i, carry2):
                cg, ce = carry2
                b = plsc.bitcast(err_v[pl.ds(i * L, L)], jnp.int32)
                tb = jnp.full((L,), t_bits, jnp.int32)
                cg = cg + jnp.sum((b > tb).astype(jnp.int32))
                ce = ce + jnp.sum((b == tb).astype(jnp.int32))
                return cg, ce
            cgt, ceq = lax.fori_loop(0, NVREG, csweep2,
                                     (jnp.int32(0), jnp.int32(0)))
            cnt_v[...] = jnp.where(lane == 0, cgt,
                                   jnp.where(lane == 1, ceq, 0))
            pltpu.sync_copy(cnt_v, cnt_sh.at[pl.ds(s * L, L)])
            plsc.subcore_barrier()
            pltpu.sync_copy(cnt_sh, cnt2d_v)

            def bases(t, carry2):
                gb, eb = carry2
                row = cnt2d_v[pl.ds(t * L, L)]
                take = (t < s).astype(jnp.int32)
                gb = gb + take * jnp.sum(jnp.where(lane == 0, row, 0))
                eb = eb + take * jnp.sum(jnp.where(lane == 1, row, 0))
                return gb, eb
            gt_base, eq_base = lax.fori_loop(0, NSUB, bases,
                                             (jnp.int32(0), jnp.int32(0)))

            def dsweep(i, carry2):
                rg, re = carry2
                b = plsc.bitcast(err_v[pl.ds(i * L, L)], jnp.int32)
                tb = jnp.full((L,), t_bits, jnp.int32)
                m_gt = b > tb
                m_eq = b == tb
                r_gt = plsc.cumsum(m_gt.astype(jnp.int32))
                r_eq = plsc.cumsum(m_eq.astype(jnp.int32))
                pos = shard + i * L + lane
                dgt = gt_base + rg + r_gt - 1
                der = eq_base + re + r_eq - 1
                kept = m_eq & (der < need)
                dest = jnp.where(m_gt, dgt,
                                 jnp.where(kept, cnt_gt_total + der, W + pos))
                dest_v[pl.ds(i * L, L)] = dest
                buf_v[pl.ds(i * L, L)] = pos
                return (rg + jnp.sum(m_gt.astype(jnp.int32)),
                        re + jnp.sum(m_eq.astype(jnp.int32)))
            lax.fori_loop(0, NVREG, dsweep, (jnp.int32(0), jnp.int32(0)))

            pltpu.async_copy(err_v, serr_hbm.at[dest_v], sem).wait()
            pltpu.async_copy(buf_v, spos_hbm.at[dest_v], sem).wait()
            pltpu.async_copy(idx_v, sidx_hbm.at[dest_v], sem).wait()

            def pfill2(i, _):
                werr_v[pl.ds(i * L, L)] = jnp.full((L,), -1.0, jnp.float32)
                wpos_v[pl.ds(i * L, L)] = N_PIX + s * TRK + i * L + lane
                widx_v[pl.ds(i * L, L)] = jnp.zeros((L,), jnp.int32)
                return 0
            lax.fori_loop(0, TRK // L, pfill2, 0)
            pltpu.sync_copy(werr_v.at[pl.ds(0, TRK)],
                            serr_hbm.at[pl.ds(K + s * TRK, TRK)])
            pltpu.sync_copy(wpos_v.at[pl.ds(0, TRK)],
                            spos_hbm.at[pl.ds(K + s * TRK, TRK)])
            pltpu.sync_copy(widx_v.at[pl.ds(0, TRK)],
                            sidx_hbm.at[pl.ds(K + s * TRK, TRK)])


def _sc_call(errors, indices, old_errors, old_indices):
    mesh = plsc.VectorSubcoreMesh(core_axis_name="c", subcore_axis_name="s",
                                  num_cores=2, num_subcores=NSUB)
    f = pl.kernel(
        _sc_body,
        mesh=mesh,
        out_type=(
            jax.ShapeDtypeStruct((K,), jnp.float32),
            jax.ShapeDtypeStruct((OUT_PAD,), jnp.float32),
            jax.ShapeDtypeStruct((OUT_PAD,), jnp.int32),
            jax.ShapeDtypeStruct((OUT_PAD,), jnp.int32),
        ),
        compiler_params=pltpu.CompilerParams(needs_layout_passes=False),
        scratch_types=[
            pltpu.VMEM_SHARED((N_PIX,), jnp.int32),
            pltpu.VMEM_SHARED((NSUB * K,), jnp.float32),
            pltpu.VMEM_SHARED((K,), jnp.float32),
            pltpu.VMEM_SHARED((NSUB * 1024,), jnp.int32),
            pltpu.VMEM_SHARED((NSUB * L,), jnp.int32),
            pltpu.VMEM((EPT,), jnp.int32),
            pltpu.VMEM((EPT,), jnp.float32),
            pltpu.VMEM((EPT,), jnp.int32),
            pltpu.VMEM((EPT,), jnp.int32),
            pltpu.VMEM((NSUB * 1024,), jnp.int32),
            pltpu.VMEM((1024,), jnp.int32),
            pltpu.VMEM((1024,), jnp.int32),
            pltpu.VMEM((K,), jnp.float32),
            pltpu.VMEM((TRK,), jnp.float32),
            pltpu.VMEM((TRK,), jnp.int32),
            pltpu.VMEM((TRK,), jnp.int32),
            pltpu.VMEM((TRK,), jnp.float32),
            pltpu.VMEM((L,), jnp.int32),
            pltpu.VMEM((NSUB * L,), jnp.int32),
            pltpu.VMEM((CAP,), jnp.float32),
            pltpu.VMEM((CAP,), jnp.int32),
            pltpu.VMEM((CAP,), jnp.int32),
            pltpu.SemaphoreType.DMA,
        ],
    )
    return f(errors, indices, old_errors, old_indices)


def _lin_iota(nr):
    r = lax.broadcasted_iota(jnp.int32, (nr, C), 0)
    c = lax.broadcasted_iota(jnp.int32, (nr, C), 1)
    return r * C + c


def _xor_shuffle(x, d, nr):
    if d < C:
        fwd = pltpu.roll(x, C - d, 1)
        bwd = pltpu.roll(x, d, 1)
        sel = (lax.broadcasted_iota(jnp.int32, (nr, C), 1) & d) == 0
    else:
        dr = d // C
        fwd = pltpu.roll(x, nr - dr, 0)
        bwd = pltpu.roll(x, dr, 0)
        sel = (lax.broadcasted_iota(jnp.int32, (nr, C), 0) & dr) == 0
    return jnp.where(sel, fwd, bwd)


def _bitonic3(key, pos, idx, nr, levels):
    e = _lin_iota(nr)
    for k in range(1, levels + 1):
        for j in range(k - 1, -1, -1):
            d = 1 << j
            kk = _xor_shuffle(key, d, nr)
            pp = _xor_shuffle(pos, d, nr)
            ii = _xor_shuffle(idx, d, nr)
            upper = (e & d) != 0
            if k < levels:
                asc = (e & (1 << k)) != 0
            else:
                asc = jnp.zeros_like(upper)
            keep_larger = ~(upper ^ asc)
            mine_gt = (key > kk) | ((key == kk) & (pos < pp))
            take_mine = ~(keep_larger ^ mine_gt)
            key = jnp.where(take_mine, key, kk)
            pos = jnp.where(take_mine, pos, pp)
            idx = jnp.where(take_mine, idx, ii)
    return key, pos, idx


def _bitonic1(v, nr, levels):
    e = _lin_iota(nr)
    for k in range(1, levels + 1):
        for j in range(k - 1, -1, -1):
            d = 1 << j
            vv = _xor_shuffle(v, d, nr)
            upper = (e & d) != 0
            if k < levels:
                asc = (e & (1 << k)) != 0
            else:
                asc = jnp.zeros_like(upper)
            keep_larger = ~(upper ^ asc)
            v = jnp.where(keep_larger, jnp.maximum(v, vv), jnp.minimum(v, vv))
    return v


def _sort_merge_body(se_ref, sp_ref, si_ref, upd_ref, oi_ref, oe_ref, ooi_ref):
    key, pos, idx = _bitonic3(se_ref[...], sp_ref[...], si_ref[...], R2, 14)
    key = key[:R, :]
    idx = idx[:R, :]
    me = _bitonic1(upd_ref[...], R, 13)
    surpassed = key > me
    oe_ref[...] = jnp.where(surpassed, key, me)
    ooi_ref[...] = jnp.where(surpassed, idx, oi_ref[...])


def _tc_call(sel_err, sel_pos, sel_idx, updated, old_idx):
    return pl.pallas_call(
        _sort_merge_body,
        out_shape=(
            jax.ShapeDtypeStruct((R, C), jnp.float32),
            jax.ShapeDtypeStruct((R, C), jnp.int32),
        ),
    )(sel_err, sel_pos, sel_idx, updated, old_idx)


def kernel(errors, indices, old_errors, old_indices):
    errors_flat = errors.reshape(-1)
    indices_flat = indices.reshape(-1)
    upd, sel_err, sel_pos, sel_idx = _sc_call(
        errors_flat, indices_flat, old_errors, old_indices)
    oe, oi = _tc_call(
        sel_err[:W].reshape(R2, C), sel_pos[:W].reshape(R2, C),
        sel_idx[:W].reshape(R2, C), upd.reshape(R, C),
        old_indices.reshape(R, C))
    return oe.reshape(K), oi.reshape(K)
